# Initial kernel scaffold; baseline (speedup 1.0000x reference)
import jax
import jax.numpy as jnp
from jax.experimental import pallas as pl


def kernel(x, edge_index, batch, W1a, b1a, W1b, b1b, g1, be1, rm1, rv1, W2a, b2a, W2b, b2b, g2, be2, rm2, rv2, Wf1, bf1, Wf2, bf2):
    raise NotImplementedError("write your pallas kernel here")


# speedup vs baseline: 5.3529x; 5.3529x over previous
import functools

import jax
import jax.numpy as jnp
from jax import lax
from jax.experimental import pallas as pl
from jax.experimental.pallas import tpu as pltpu
from jax.experimental.pallas import tpu_sc as plsc

_N = 10000
_E = 320000
_DIN = 128
_DIM = 64
_DOUT = 10
_G = 64

_NC = 2
_NS = 16
_NW = _NC * _NS
_NPAD = 10240
_RPS = _NPAD // _NS
_EPW = _E // _NW
_K = 80
_CHUNKS = _EPW // _K


def _segsum_sc(table, src, dst, zeros_blk):
  mesh = plsc.VectorSubcoreMesh(core_axis_name="c", subcore_axis_name="s")

  @functools.partial(
      pl.kernel,
      out_type=jax.ShapeDtypeStruct((_NC, _NPAD, _DIM), jnp.float32),
      mesh=mesh,
      scratch_types=[
          pltpu.VMEM((_K,), jnp.int32),
          pltpu.VMEM((_K,), jnp.int32),
          pltpu.VMEM((_K, _DIM), jnp.float32),
          pltpu.VMEM_SHARED((_NPAD, _DIM), jnp.float32),
          pltpu.SemaphoreType.DMA,
      ],
      compiler_params=pltpu.CompilerParams(use_tc_tiling_on_sc=False),
  )
  def k(table_hbm, src_hbm, dst_hbm, z_hbm, out_hbm, sidx, didx, rows, accum,
        sem):
    c = lax.axis_index("c")
    s = lax.axis_index("s")
    wid = c * _NS + s

    pltpu.sync_copy(z_hbm, accum.at[pl.ds(s * _RPS, _RPS)])
    plsc.subcore_barrier()

    base0 = wid * _EPW

    @pl.loop(0, _CHUNKS)
    def _(i):
      base = base0 + i * _K
      pltpu.sync_copy(src_hbm.at[pl.ds(base, _K)], sidx)
      pltpu.sync_copy(dst_hbm.at[pl.ds(base, _K)], didx)
      pltpu.async_copy(table_hbm.at[sidx], rows, sem).wait()
      pltpu.sync_copy(rows, accum.at[didx], add=True)

    plsc.subcore_barrier()
    pltpu.sync_copy(accum.at[pl.ds(s * _RPS, _RPS)],
                    out_hbm.at[c, pl.ds(s * _RPS, _RPS)])

  return k(table, src, dst, zeros_blk)


def _proj_body(x_ref, w_ref, o_ref):
  o_ref[...] = jnp.dot(x_ref[...], w_ref[...],
                       preferred_element_type=jnp.float32)


def _mid_body(agg_ref, y_ref, b1a_ref, w1b_ref, b1b_ref, g1_ref, be1_ref,
              rm1_ref, rv1_ref, w2a_ref, z_ref):
  agg = agg_ref[0, :_N, :] + agg_ref[1, :_N, :]
  t = jax.nn.relu(agg + y_ref[...] + b1a_ref[...])
  h = jnp.dot(t, w1b_ref[...], preferred_element_type=jnp.float32)
  h = jax.nn.relu(h + b1b_ref[...])
  h = (h - rm1_ref[...]) / jnp.sqrt(rv1_ref[...] + 1e-5) * g1_ref[...] \
      + be1_ref[...]
  z_ref[...] = jnp.dot(h, w2a_ref[...], preferred_element_type=jnp.float32)


def _tail_body(agg_ref, z_ref, b2a_ref, w2b_ref, b2b_ref, g2_ref, be2_ref,
               rm2_ref, rv2_ref, batch_ref, wf1_ref, bf1_ref, wf2_ref,
               bf2_ref, o_ref):
  agg = agg_ref[0, :_N, :] + agg_ref[1, :_N, :]
  t = jax.nn.relu(agg + z_ref[...] + b2a_ref[...])
  h2 = jnp.dot(t, w2b_ref[...], preferred_element_type=jnp.float32)
  h2 = jax.nn.relu(h2 + b2b_ref[...])
  h2 = (h2 - rm2_ref[...]) / jnp.sqrt(rv2_ref[...] + 1e-5) * g2_ref[...] \
      + be2_ref[...]

  seg = (lax.broadcasted_iota(jnp.int32, (_G, _N), 0)
         == batch_ref[...]).astype(jnp.float32)
  pooled = jnp.dot(seg, h2, preferred_element_type=jnp.float32)
  counts = jnp.maximum(jnp.sum(seg, axis=1, keepdims=True), 1.0)
  pooled = pooled / counts

  h3 = jax.nn.relu(
      jnp.dot(pooled, wf1_ref[...], preferred_element_type=jnp.float32)
      + bf1_ref[...])
  o_ref[...] = jnp.dot(h3, wf2_ref[...],
                       preferred_element_type=jnp.float32) + bf2_ref[...]


def kernel(x, edge_index, batch, W1a, b1a, W1b, b1b, g1, be1, rm1, rv1,
           W2a, b2a, W2b, b2b, g2, be2, rm2, rv2, Wf1, bf1, Wf2, bf2):
  f32 = jnp.float32
  zeros_blk = jnp.zeros((_RPS, _DIM), f32)
  r = lambda v: v.reshape(1, -1)

  y = pl.pallas_call(
      _proj_body,
      out_shape=jax.ShapeDtypeStruct((_N, _DIM), f32),
  )(x, W1a)

  src = edge_index[0]
  dst = edge_index[1]
  agg1 = _segsum_sc(y, src, dst, zeros_blk)

  z = pl.pallas_call(
      _mid_body,
      out_shape=jax.ShapeDtypeStruct((_N, _DIM), f32),
  )(agg1, y, r(b1a), W1b, r(b1b), r(g1), r(be1), r(rm1), r(rv1), W2a)

  agg2 = _segsum_sc(z, src, dst, zeros_blk)

  out = pl.pallas_call(
      _tail_body,
      out_shape=jax.ShapeDtypeStruct((_G, _DOUT), f32),
  )(agg2, z, r(b2a), W2b, r(b2b), r(g2), r(be2), r(rm2), r(rv2),
    batch.reshape(1, _N), Wf1, r(bf1), Wf2, r(bf2))

  return out
